# Initial kernel scaffold; baseline (speedup 1.0000x reference)
import functools

import jax
import jax.numpy as jnp
from jax import lax
from jax.experimental import pallas as pl

ALPHA = 0.2
N_NODE = 10000
N_EDGE = 1024
D = 128

TN1 = 1000
TE2 = 256
TE3 = 128

f32 = jnp.float32


def _lrelu(t):
    return jnp.where(t > 0, t, ALPHA * t)


def _prep_body(has_w, *refs):
    if has_w:
        (x_ref, w2_ref, a_ref, a2_ref, wc_ref, w_ref,
         xa_ref, s_ref, q_ref, smax_ref, qmax_ref, xv_ref) = refs
    else:
        (x_ref, w2_ref, a_ref, a2_ref, wc_ref,
         xa_ref, s_ref, q_ref, smax_ref, qmax_ref) = refs
    x = x_ref[...]
    xa = jnp.dot(x, w2_ref[...], preferred_element_type=f32)
    c = jnp.sum(wc_ref[...] * a_ref[0:1, :])
    s = _lrelu(c + lax.dot_general(xa, a_ref[1:2, :],
                                   (((1,), (1,)), ((), ())),
                                   preferred_element_type=f32))
    q = lax.dot_general(xa, a2_ref[0:1, :], (((1,), (1,)), ((), ())),
                        preferred_element_type=f32)
    xa_ref[...] = xa
    s_ref[...] = s
    q_ref[...] = q
    smax_ref[...] = jnp.max(s).reshape(1, 1)
    qmax_ref[...] = jnp.max(q).reshape(1, 1)
    if has_w:
        xv_ref[...] = jnp.dot(x, w_ref[...], preferred_element_type=f32)


def _edge_body(h_ref, s_ref, smax_ref, xv_ref, w3_ref, a2_ref,
               edge_ref, y_ref, ymax_ref):
    expw = jnp.exp(s_ref[...] - smax_ref[0, 0])
    w = h_ref[...] * expw
    num = jnp.dot(w, xv_ref[...], preferred_element_type=f32)
    den = jnp.sum(w, axis=1, keepdims=True)
    mx = jnp.sum(xv_ref[...], axis=0, keepdims=True) * (1.0 / N_NODE)
    edge = jnp.where(den > 0, num / den, mx)
    edge_ref[...] = edge
    w3a = lax.dot_general(w3_ref[...], a2_ref[1:2, :],
                          (((1,), (1,)), ((), ())), preferred_element_type=f32)
    y = jnp.dot(edge, w3a, preferred_element_type=f32)
    y_ref[...] = y
    ymax_ref[...] = jnp.max(y).reshape(1, 1)


def _node_body(h_ref, q_ref, y_ref, edge_ref, m2_ref, out_ref):
    @pl.when(pl.program_id(0) == 0)
    def _():
        out_ref[...] = jnp.zeros_like(out_ref)

    z = _lrelu(y_ref[...] + q_ref[...])
    w = h_ref[...] * jnp.exp(z - m2_ref[0, 0])
    aug = jnp.concatenate(
        [edge_ref[...], jnp.ones((TE3, D), f32)], axis=1)
    out_ref[...] += lax.dot_general(w, aug, (((0,), (0,)), ((), ())),
                                    preferred_element_type=f32)


def _norm_body(concat, aug_ref, edge_ref, out_ref):
    aug = aug_ref[...]
    num = aug[:, :D]
    den = aug[:, D:D + 1]
    emean = jnp.sum(edge_ref[...], axis=0, keepdims=True) * (1.0 / N_EDGE)
    node = jnp.where(den > 0, num / den, emean)
    if concat:
        node = jnp.where(node > 0, node, jnp.exp(node) - 1.0)
    out_ref[...] = node


def _layer(x, H, w, w2, w3, wc, a, a2, concat):
    n, e, d = N_NODE, N_EDGE, D
    has_w = w is not None
    a_r = a.reshape(2, d)
    a2_r = a2.reshape(2, d)
    wc_r = wc.reshape(1, d)

    full = lambda shape: pl.BlockSpec(shape, lambda i: (0, 0))
    rows = lambda t: pl.BlockSpec(t, lambda i: (i, 0))

    prep_ins = [x, w2, a_r, a2_r, wc_r] + ([w] if has_w else [])
    prep_in_specs = [rows((TN1, d)), full((d, d)), full((2, d)),
                     full((2, d)), full((1, d))] + ([full((d, d))] if has_w else [])
    prep_outs = [
        jax.ShapeDtypeStruct((n, d), f32),
        jax.ShapeDtypeStruct((n, 1), f32),
        jax.ShapeDtypeStruct((n, 1), f32),
        jax.ShapeDtypeStruct((n // TN1, 1), f32),
        jax.ShapeDtypeStruct((n // TN1, 1), f32),
    ] + ([jax.ShapeDtypeStruct((n, d), f32)] if has_w else [])
    prep_out_specs = [rows((TN1, d)), rows((TN1, 1)), rows((TN1, 1)),
                      rows((1, 1)), rows((1, 1))] + ([rows((TN1, d))] if has_w else [])
    res = pl.pallas_call(
        functools.partial(_prep_body, has_w),
        grid=(n // TN1,),
        in_specs=prep_in_specs,
        out_specs=prep_out_specs,
        out_shape=prep_outs,
    )(*prep_ins)
    if has_w:
        xa, s, q, smax_p, qmax_p, xv = res
    else:
        xa, s, q, smax_p, qmax_p = res
        xv = x

    smax = jnp.max(smax_p).reshape(1, 1)
    s_row = s.reshape(1, n)
    q_row = q.reshape(1, n)

    edge, y, ymax_p = pl.pallas_call(
        _edge_body,
        grid=(e // TE2,),
        in_specs=[rows((TE2, n)), full((1, n)), full((1, 1)),
                  full((n, d)), full((d, d)), full((2, d))],
        out_specs=[rows((TE2, d)), rows((TE2, 1)), rows((1, 1))],
        out_shape=[jax.ShapeDtypeStruct((e, d), f32),
                   jax.ShapeDtypeStruct((e, 1), f32),
                   jax.ShapeDtypeStruct((e // TE2, 1), f32)],
    )(H, s_row, smax, xv, w3, a2_r)

    m2 = _lrelu(jnp.max(qmax_p) + jnp.max(ymax_p)).reshape(1, 1)

    aug = pl.pallas_call(
        _node_body,
        grid=(e // TE3,),
        in_specs=[rows((TE3, n)), full((1, n)), rows((TE3, 1)),
                  rows((TE3, d)), full((1, 1))],
        out_specs=full((n, 2 * d)),
        out_shape=jax.ShapeDtypeStruct((n, 2 * d), f32),
    )(H, q_row, y, edge, m2)

    node = pl.pallas_call(
        functools.partial(_norm_body, concat),
        grid=(n // TN1,),
        in_specs=[rows((TN1, 2 * d)), full((e, d))],
        out_specs=rows((TN1, d)),
        out_shape=jax.ShapeDtypeStruct((n, d), f32),
    )(aug, edge)
    return node


@jax.jit
def kernel(x, H, g1_w2, g1_w3, g1_wc, g1_a, g1_a2,
           g2_w, g2_w2, g2_w3, g2_wc, g2_a, g2_a2):
    x2 = x[0]
    H2 = H[0]
    h = _layer(x2, H2, None, g1_w2, g1_w3, g1_wc, g1_a, g1_a2, concat=True)
    out = _layer(h, H2, g2_w, g2_w2, g2_w3, g2_wc, g2_a, g2_a2, concat=False)
    return out.reshape(1, N_NODE, D)


# speedup vs baseline: 1.0401x; 1.0401x over previous
import functools

import jax
import jax.numpy as jnp
from jax import lax
from jax.experimental import pallas as pl

ALPHA = 0.2
N_NODE = 10000
N_EDGE = 1024
D = 128

TN1 = 1000
TE2 = 256
TE3 = 128

f32 = jnp.float32


def _lrelu(t):
    return jnp.where(t > 0, t, ALPHA * t)


def _prep_body(has_w, *refs):
    if has_w:
        (x_ref, w2_ref, a_ref, a2_ref, wc_ref, w_ref,
         xa_ref, s_ref, q_ref, smax_ref, qmax_ref, xv_ref) = refs
    else:
        (x_ref, w2_ref, a_ref, a2_ref, wc_ref,
         xa_ref, s_ref, q_ref, smax_ref, qmax_ref) = refs
    x = x_ref[...]
    xa = jnp.dot(x, w2_ref[...], preferred_element_type=f32)
    c = jnp.sum(wc_ref[...] * a_ref[0:1, :])
    s = _lrelu(c + lax.dot_general(xa, a_ref[1:2, :],
                                   (((1,), (1,)), ((), ())),
                                   preferred_element_type=f32))
    q = lax.dot_general(xa, a2_ref[0:1, :], (((1,), (1,)), ((), ())),
                        preferred_element_type=f32)
    xa_ref[...] = xa
    s_ref[...] = s
    q_ref[...] = q
    smax_ref[...] = jnp.max(s).reshape(1, 1, 1)
    qmax_ref[...] = jnp.max(q).reshape(1, 1, 1)
    if has_w:
        xv_ref[...] = jnp.dot(x, w_ref[...], preferred_element_type=f32)


def _edge_body(h_ref, s_ref, smax_ref, xv_ref, w3_ref, a2_ref,
               edge_ref, y_ref, ymax_ref):
    expw = jnp.exp(s_ref[...] - smax_ref[0, 0])
    w = h_ref[...] * expw
    num = jnp.dot(w, xv_ref[...], preferred_element_type=f32)
    den = jnp.sum(w, axis=1, keepdims=True)
    mx = jnp.sum(xv_ref[...], axis=0, keepdims=True) * (1.0 / N_NODE)
    edge = jnp.where(den > 0, num / den, mx)
    edge_ref[...] = edge
    w3a = lax.dot_general(w3_ref[...], a2_ref[1:2, :],
                          (((1,), (1,)), ((), ())), preferred_element_type=f32)
    y = jnp.dot(edge, w3a, preferred_element_type=f32)
    y_ref[...] = y
    ymax_ref[...] = jnp.max(y).reshape(1, 1, 1)


def _node_body(h_ref, q_ref, y_ref, edge_ref, m2_ref, out_ref):
    @pl.when(pl.program_id(0) == 0)
    def _():
        out_ref[...] = jnp.zeros_like(out_ref)

    z = _lrelu(y_ref[...] + q_ref[...])
    w = h_ref[...] * jnp.exp(z - m2_ref[0, 0])
    aug = jnp.concatenate(
        [edge_ref[...], jnp.ones((TE3, D), f32)], axis=1)
    out_ref[...] += lax.dot_general(w, aug, (((0,), (0,)), ((), ())),
                                    preferred_element_type=f32)


def _norm_body(concat, aug_ref, edge_ref, out_ref):
    aug = aug_ref[...]
    num = aug[:, :D]
    den = aug[:, D:D + 1]
    emean = jnp.sum(edge_ref[...], axis=0, keepdims=True) * (1.0 / N_EDGE)
    node = jnp.where(den > 0, num / den, emean)
    if concat:
        node = jnp.where(node > 0, node, jnp.exp(node) - 1.0)
    out_ref[...] = node


def _layer(x, H, w, w2, w3, wc, a, a2, concat):
    n, e, d = N_NODE, N_EDGE, D
    has_w = w is not None
    a_r = a.reshape(2, d)
    a2_r = a2.reshape(2, d)
    wc_r = wc.reshape(1, d)

    full = lambda shape: pl.BlockSpec(shape, lambda i: (0, 0))
    rows = lambda t: pl.BlockSpec(t, lambda i: (i, 0))

    prep_ins = [x, w2, a_r, a2_r, wc_r] + ([w] if has_w else [])
    prep_in_specs = [rows((TN1, d)), full((d, d)), full((2, d)),
                     full((2, d)), full((1, d))] + ([full((d, d))] if has_w else [])
    prep_outs = [
        jax.ShapeDtypeStruct((n, d), f32),
        jax.ShapeDtypeStruct((n, 1), f32),
        jax.ShapeDtypeStruct((n, 1), f32),
        jax.ShapeDtypeStruct((n // TN1, 1, 1), f32),
        jax.ShapeDtypeStruct((n // TN1, 1, 1), f32),
    ] + ([jax.ShapeDtypeStruct((n, d), f32)] if has_w else [])
    pmax = pl.BlockSpec((1, 1, 1), lambda i: (i, 0, 0))
    prep_out_specs = [rows((TN1, d)), rows((TN1, 1)), rows((TN1, 1)),
                      pmax, pmax] + ([rows((TN1, d))] if has_w else [])
    res = pl.pallas_call(
        functools.partial(_prep_body, has_w),
        grid=(n // TN1,),
        in_specs=prep_in_specs,
        out_specs=prep_out_specs,
        out_shape=prep_outs,
    )(*prep_ins)
    if has_w:
        xa, s, q, smax_p, qmax_p, xv = res
    else:
        xa, s, q, smax_p, qmax_p = res
        xv = x

    smax = jnp.max(smax_p).reshape(1, 1)
    s_row = s.reshape(1, n)
    q_row = q.reshape(1, n)

    edge, y, ymax_p = pl.pallas_call(
        _edge_body,
        grid=(e // TE2,),
        in_specs=[rows((TE2, n)), full((1, n)), full((1, 1)),
                  full((n, d)), full((d, d)), full((2, d))],
        out_specs=[rows((TE2, d)), rows((TE2, 1)),
                   pl.BlockSpec((1, 1, 1), lambda i: (i, 0, 0))],
        out_shape=[jax.ShapeDtypeStruct((e, d), f32),
                   jax.ShapeDtypeStruct((e, 1), f32),
                   jax.ShapeDtypeStruct((e // TE2, 1, 1), f32)],
    )(H, s_row, smax, xv, w3, a2_r)

    m2 = _lrelu(jnp.max(qmax_p) + jnp.max(ymax_p)).reshape(1, 1)

    aug = pl.pallas_call(
        _node_body,
        grid=(e // TE3,),
        in_specs=[rows((TE3, n)), full((1, n)), rows((TE3, 1)),
                  rows((TE3, d)), full((1, 1))],
        out_specs=full((n, 2 * d)),
        out_shape=jax.ShapeDtypeStruct((n, 2 * d), f32),
    )(H, q_row, y, edge, m2)

    node = pl.pallas_call(
        functools.partial(_norm_body, concat),
        grid=(n // TN1,),
        in_specs=[rows((TN1, 2 * d)), full((e, d))],
        out_specs=rows((TN1, d)),
        out_shape=jax.ShapeDtypeStruct((n, d), f32),
    )(aug, edge)
    return node


@jax.jit
def kernel(x, H, g1_w2, g1_w3, g1_wc, g1_a, g1_a2,
           g2_w, g2_w2, g2_w3, g2_wc, g2_a, g2_a2):
    x2 = x[0]
    H2 = H[0]
    h = _layer(x2, H2, None, g1_w2, g1_w3, g1_wc, g1_a, g1_a2, concat=True)
    out = _layer(h, H2, g2_w, g2_w2, g2_w3, g2_wc, g2_a, g2_a2, concat=False)
    return out.reshape(1, N_NODE, D)


# speedup vs baseline: 1.2013x; 1.1550x over previous
import functools

import jax
import jax.numpy as jnp
from jax import lax
from jax.experimental import pallas as pl

ALPHA = 0.2
N_NODE = 10000
N_EDGE = 1024
D = 128

TN1 = 1000
TE2 = 256
TE3 = 128

f32 = jnp.float32
bf16 = jnp.bfloat16


def _lrelu(t):
    return jnp.where(t > 0, t, ALPHA * t)


def _row_dot(vec2d, mat):
    return lax.dot_general(vec2d, mat, (((1,), (1,)), ((), ())),
                           preferred_element_type=f32)


def _cast_body(h_ref, hb_ref):
    hb_ref[...] = h_ref[...].astype(bf16)


def _prep_body(x_ref, w2_ref, a_ref, a2_ref, wc_ref,
               s_ref, q_ref, smax_ref, qmax_ref):
    x = x_ref[...]
    xa = jnp.dot(x, w2_ref[...], preferred_element_type=f32)
    c = jnp.sum(wc_ref[...] * a_ref[0:1, :])
    s = _lrelu(c + _row_dot(a_ref[1:2, :], xa))
    q = _row_dot(a2_ref[0:1, :], xa)
    s_ref[...] = s.reshape(1, 1, TN1)
    q_ref[...] = q.reshape(1, 1, TN1)
    smax_ref[...] = jnp.max(s).reshape(1, 1, 1)
    qmax_ref[...] = jnp.max(q).reshape(1, 1, 1)


def _edge_body(hb_ref, s_ref, smax_ref, xvb_ref, w3_ref, a2_ref,
               edge_ref, y_ref, ymax_ref):
    expw = jnp.exp(s_ref[...] - smax_ref[0, 0]).astype(bf16)
    w = hb_ref[...] * expw
    xvb = xvb_ref[...]
    num = jnp.dot(w, xvb, preferred_element_type=f32)
    den = jnp.sum(w.astype(f32), axis=1, keepdims=True)
    mx = jnp.sum(xvb.astype(f32), axis=0, keepdims=True) * (1.0 / N_NODE)
    edge = jnp.where(den > 0, num / den, mx)
    edge_ref[...] = edge
    w3a = lax.dot_general(w3_ref[...], a2_ref[1:2, :],
                          (((1,), (1,)), ((), ())), preferred_element_type=f32)
    y = jnp.dot(edge, w3a, preferred_element_type=f32)
    y_ref[...] = y
    ymax_ref[...] = jnp.max(y).reshape(1, 1, 1)


def _node_body(hb_ref, q_ref, qmax_ref, y_ref, ymax_ref, edge_ref, out_ref):
    @pl.when(pl.program_id(0) == 0)
    def _():
        out_ref[...] = jnp.zeros_like(out_ref)

    q = q_ref[...]
    qm = qmax_ref[0, 0]
    ym = ymax_ref[0, 0]
    a_row = jnp.exp(q - qm).astype(bf16)
    c_row = jnp.exp(ALPHA * q - qm).astype(bf16)
    y = y_ref[...]
    b_col = jnp.exp(y - ym).astype(bf16)
    d_col = jnp.exp(ALPHA * y - ym).astype(bf16)
    w = hb_ref[...] * jnp.maximum(b_col * a_row, d_col * c_row)
    aug = jnp.concatenate(
        [edge_ref[...].astype(bf16), jnp.ones((TE3, D), bf16)], axis=1)
    out_ref[...] += lax.dot_general(w, aug, (((0,), (0,)), ((), ())),
                                    preferred_element_type=f32)


def _norm_prep_body(aug_ref, edge_ref, w2_ref, w_ref, a_ref, a2_ref, wc_ref,
                    xvb_ref, s_ref, q_ref, smax_ref, qmax_ref):
    aug = aug_ref[...]
    num = aug[:, :D]
    den = aug[:, D:D + 1]
    emean = jnp.sum(edge_ref[...], axis=0, keepdims=True) * (1.0 / N_EDGE)
    node = jnp.where(den > 0, num / den, emean)
    h = jnp.where(node > 0, node, jnp.exp(node) - 1.0)
    xa = jnp.dot(h, w2_ref[...], preferred_element_type=f32)
    c = jnp.sum(wc_ref[...] * a_ref[0:1, :])
    s = _lrelu(c + _row_dot(a_ref[1:2, :], xa))
    q = _row_dot(a2_ref[0:1, :], xa)
    xvb_ref[...] = jnp.dot(h, w_ref[...], preferred_element_type=f32).astype(bf16)
    s_ref[...] = s.reshape(1, 1, TN1)
    q_ref[...] = q.reshape(1, 1, TN1)
    smax_ref[...] = jnp.max(s).reshape(1, 1, 1)
    qmax_ref[...] = jnp.max(q).reshape(1, 1, 1)


def _norm_body(aug_ref, edge_ref, out_ref):
    aug = aug_ref[...]
    num = aug[:, :D]
    den = aug[:, D:D + 1]
    emean = jnp.sum(edge_ref[...], axis=0, keepdims=True) * (1.0 / N_EDGE)
    out_ref[...] = jnp.where(den > 0, num / den, emean)


def _full(shape):
    nd = len(shape)
    return pl.BlockSpec(shape, lambda i: (0,) * nd)


def _rows(t):
    nd = len(t)
    return pl.BlockSpec(t, lambda i: (i,) + (0,) * (nd - 1))


def _edge_pass(Hb, s3, smax_p, xvb, w3, a2_r):
    n, e, d = N_NODE, N_EDGE, D
    s_row = s3.reshape(1, n)
    smax = jnp.max(smax_p).reshape(1, 1)
    return pl.pallas_call(
        _edge_body,
        grid=(e // TE2,),
        in_specs=[_rows((TE2, n)), _full((1, n)), _full((1, 1)),
                  _full((n, d)), _full((d, d)), _full((2, d))],
        out_specs=[_rows((TE2, d)), _rows((TE2, 1)), _rows((1, 1, 1))],
        out_shape=[jax.ShapeDtypeStruct((e, d), f32),
                   jax.ShapeDtypeStruct((e, 1), f32),
                   jax.ShapeDtypeStruct((e // TE2, 1, 1), f32)],
    )(Hb, s_row, smax, xvb, w3, a2_r)


def _node_pass(Hb, q3, qmax_p, y, ymax_p, edge):
    n, e, d = N_NODE, N_EDGE, D
    q_row = q3.reshape(1, n)
    qmax = jnp.max(qmax_p).reshape(1, 1)
    ymax = jnp.max(ymax_p).reshape(1, 1)
    return pl.pallas_call(
        _node_body,
        grid=(e // TE3,),
        in_specs=[_rows((TE3, n)), _full((1, n)), _full((1, 1)),
                  _rows((TE3, 1)), _full((1, 1)), _rows((TE3, d))],
        out_specs=_full((n, 2 * d)),
        out_shape=jax.ShapeDtypeStruct((n, 2 * d), f32),
    )(Hb, q_row, qmax, y, ymax, edge)


@jax.jit
def kernel(x, H, g1_w2, g1_w3, g1_wc, g1_a, g1_a2,
           g2_w, g2_w2, g2_w3, g2_wc, g2_a, g2_a2):
    n, e, d = N_NODE, N_EDGE, D
    x2 = x[0]
    H2 = H[0]
    a1_r = g1_a.reshape(2, d)
    a21_r = g1_a2.reshape(2, d)
    wc1_r = g1_wc.reshape(1, d)
    a2_r = g2_a.reshape(2, d)
    a22_r = g2_a2.reshape(2, d)
    wc2_r = g2_wc.reshape(1, d)

    Hb = pl.pallas_call(
        _cast_body,
        grid=(e // TE3,),
        in_specs=[_rows((TE3, n))],
        out_specs=_rows((TE3, n)),
        out_shape=jax.ShapeDtypeStruct((e, n), bf16),
    )(H2)

    s3, q3, smax_p, qmax_p = pl.pallas_call(
        _prep_body,
        grid=(n // TN1,),
        in_specs=[_rows((TN1, d)), _full((d, d)), _full((2, d)),
                  _full((2, d)), _full((1, d))],
        out_specs=[_rows((1, 1, TN1)), _rows((1, 1, TN1)),
                   _rows((1, 1, 1)), _rows((1, 1, 1))],
        out_shape=[jax.ShapeDtypeStruct((n // TN1, 1, TN1), f32),
                   jax.ShapeDtypeStruct((n // TN1, 1, TN1), f32),
                   jax.ShapeDtypeStruct((n // TN1, 1, 1), f32),
                   jax.ShapeDtypeStruct((n // TN1, 1, 1), f32)],
    )(x2, g1_w2, a1_r, a21_r, wc1_r)

    xvb1 = x2.astype(bf16)
    edge1, y1, ymax_p1 = _edge_pass(Hb, s3, smax_p, xvb1, g1_w3, a21_r)
    aug1 = _node_pass(Hb, q3, qmax_p, y1, ymax_p1, edge1)

    xvb2, s3b, q3b, smax_pb, qmax_pb = pl.pallas_call(
        _norm_prep_body,
        grid=(n // TN1,),
        in_specs=[_rows((TN1, 2 * d)), _full((e, d)), _full((d, d)),
                  _full((d, d)), _full((2, d)), _full((2, d)), _full((1, d))],
        out_specs=[_rows((TN1, d)), _rows((1, 1, TN1)), _rows((1, 1, TN1)),
                   _rows((1, 1, 1)), _rows((1, 1, 1))],
        out_shape=[jax.ShapeDtypeStruct((n, d), bf16),
                   jax.ShapeDtypeStruct((n // TN1, 1, TN1), f32),
                   jax.ShapeDtypeStruct((n // TN1, 1, TN1), f32),
                   jax.ShapeDtypeStruct((n // TN1, 1, 1), f32),
                   jax.ShapeDtypeStruct((n // TN1, 1, 1), f32)],
    )(aug1, edge1, g2_w2, g2_w, a2_r, a22_r, wc2_r)

    edge2, y2, ymax_p2 = _edge_pass(Hb, s3b, smax_pb, xvb2, g2_w3, a22_r)
    aug2 = _node_pass(Hb, q3b, qmax_pb, y2, ymax_p2, edge2)

    out = pl.pallas_call(
        _norm_body,
        grid=(n // TN1,),
        in_specs=[_rows((TN1, 2 * d)), _full((e, d))],
        out_specs=_rows((TN1, d)),
        out_shape=jax.ShapeDtypeStruct((n, d), f32),
    )(aug2, edge2)
    return out.reshape(1, n, d)


# speedup vs baseline: 1.2873x; 1.0716x over previous
import jax
import jax.numpy as jnp
from jax import lax
from jax.experimental import pallas as pl

ALPHA = 0.2
N_NODE = 10000
N_EDGE = 1024
D = 128

TE2 = 256
TE3 = 128

f32 = jnp.float32
bf16 = jnp.bfloat16


def _lrelu(t):
    return jnp.where(t > 0, t, ALPHA * t)


def _sq(xa, a_ref, a2_ref, wc_ref):
    a_top = a_ref[0:D, :]
    a_bot = a_ref[D:2 * D, :]
    c = jnp.dot(wc_ref[...], a_top, preferred_element_type=f32)
    dot_rows = lambda v: lax.dot_general(v, xa, (((0,), (1,)), ((), ())),
                                         preferred_element_type=f32)
    s = _lrelu(c[0, 0] + dot_rows(a_bot))
    q = dot_rows(a2_ref[0:D, :])
    return s, q


def _cast_body(h_ref, hb_ref):
    hb_ref[...] = h_ref[...].astype(bf16)


def _prep_body(x_ref, w2_ref, a_ref, a2_ref, wc_ref,
               s_ref, q_ref, smax_ref, qmax_ref):
    xa = jnp.dot(x_ref[...], w2_ref[...], preferred_element_type=f32)
    s, q = _sq(xa, a_ref, a2_ref, wc_ref)
    s_ref[...] = s
    q_ref[...] = q
    smax_ref[...] = jnp.max(s).reshape(1, 1)
    qmax_ref[...] = jnp.max(q).reshape(1, 1)


def _edge_body(hb_ref, s_ref, smax_ref, xvb_ref, w3_ref, a2_ref,
               edge_ref, y_ref):
    expw = jnp.exp(s_ref[...] - smax_ref[0, 0]).astype(bf16)
    w = hb_ref[...] * expw
    xvb = xvb_ref[...]
    num = jnp.dot(w, xvb, preferred_element_type=f32)
    den = jnp.sum(w.astype(f32), axis=1, keepdims=True)
    mx = jnp.sum(xvb.astype(f32), axis=0, keepdims=True) * (1.0 / N_NODE)
    edge = jnp.where(den > 0, num / den, mx)
    edge_ref[...] = edge
    w3a = jnp.dot(w3_ref[...], a2_ref[D:2 * D, :],
                  preferred_element_type=f32)
    y_ref[...] = jnp.dot(edge, w3a, preferred_element_type=f32)


def _node_body(hb_ref, q_ref, qmax_ref, y_ref, yfull_ref, edge_ref, out_ref):
    @pl.when(pl.program_id(0) == 0)
    def _():
        out_ref[...] = jnp.zeros_like(out_ref)

    q = q_ref[...]
    qm = qmax_ref[0, 0]
    ym = jnp.max(yfull_ref[...])
    a_row = jnp.exp(q - qm).astype(bf16)
    c_row = jnp.exp(ALPHA * q - qm).astype(bf16)
    y = y_ref[...]
    b_col = jnp.exp(y - ym).astype(bf16)
    d_col = jnp.exp(ALPHA * y - ym).astype(bf16)
    w = hb_ref[...] * jnp.maximum(b_col * a_row, d_col * c_row)
    aug = jnp.concatenate(
        [edge_ref[...].astype(bf16), jnp.ones((TE3, D), bf16)], axis=1)
    out_ref[...] += lax.dot_general(w, aug, (((0,), (0,)), ((), ())),
                                    preferred_element_type=f32)


def _norm_prep_body(aug_ref, edge_ref, w2_ref, w_ref, a_ref, a2_ref, wc_ref,
                    xvb_ref, s_ref, q_ref, smax_ref, qmax_ref):
    aug = aug_ref[...]
    num = aug[:, :D]
    den = aug[:, D:D + 1]
    emean = jnp.sum(edge_ref[...], axis=0, keepdims=True) * (1.0 / N_EDGE)
    node = jnp.where(den > 0, num / den, emean)
    h = jnp.where(node > 0, node, jnp.exp(node) - 1.0)
    xa = jnp.dot(h, w2_ref[...], preferred_element_type=f32)
    s, q = _sq(xa, a_ref, a2_ref, wc_ref)
    xvb_ref[...] = jnp.dot(h, w_ref[...], preferred_element_type=f32).astype(bf16)
    s_ref[...] = s
    q_ref[...] = q
    smax_ref[...] = jnp.max(s).reshape(1, 1)
    qmax_ref[...] = jnp.max(q).reshape(1, 1)


def _norm_body(aug_ref, edge_ref, out_ref):
    aug = aug_ref[...]
    num = aug[:, :D]
    den = aug[:, D:D + 1]
    emean = jnp.sum(edge_ref[...], axis=0, keepdims=True) * (1.0 / N_EDGE)
    out_ref[...] = jnp.where(den > 0, num / den, emean)


def _full(shape):
    nd = len(shape)
    return pl.BlockSpec(shape, lambda i: (0,) * nd)


def _rows(t):
    nd = len(t)
    return pl.BlockSpec(t, lambda i: (i,) + (0,) * (nd - 1))


def _edge_pass(Hb, s, smax, xvb, w3, a2_c):
    n, e, d = N_NODE, N_EDGE, D
    return pl.pallas_call(
        _edge_body,
        grid=(e // TE2,),
        in_specs=[_rows((TE2, n)), _full((1, n)), _full((1, 1)),
                  _full((n, d)), _full((d, d)), _full((2 * d, 1))],
        out_specs=[_rows((TE2, d)), _rows((TE2, 1))],
        out_shape=[jax.ShapeDtypeStruct((e, d), f32),
                   jax.ShapeDtypeStruct((e, 1), f32)],
    )(Hb, s, smax, xvb, w3, a2_c)


def _node_pass(Hb, q, qmax, y, edge):
    n, e, d = N_NODE, N_EDGE, D
    return pl.pallas_call(
        _node_body,
        grid=(e // TE3,),
        in_specs=[_rows((TE3, n)), _full((1, n)), _full((1, 1)),
                  _rows((TE3, 1)), _full((e, 1)), _rows((TE3, d))],
        out_specs=_full((n, 2 * d)),
        out_shape=jax.ShapeDtypeStruct((n, 2 * d), f32),
    )(Hb, q, qmax, y, y, edge)


@jax.jit
def kernel(x, H, g1_w2, g1_w3, g1_wc, g1_a, g1_a2,
           g2_w, g2_w2, g2_w3, g2_wc, g2_a, g2_a2):
    n, e, d = N_NODE, N_EDGE, D
    x2 = x[0]
    H2 = H[0]
    wc1_r = g1_wc.reshape(1, d)
    wc2_r = g2_wc.reshape(1, d)

    Hb = pl.pallas_call(
        _cast_body,
        grid=(e // TE3,),
        in_specs=[_rows((TE3, n))],
        out_specs=_rows((TE3, n)),
        out_shape=jax.ShapeDtypeStruct((e, n), bf16),
    )(H2)

    s1, q1, smax1, qmax1 = pl.pallas_call(
        _prep_body,
        grid=(1,),
        in_specs=[_full((n, d)), _full((d, d)), _full((2 * d, 1)),
                  _full((2 * d, 1)), _full((1, d))],
        out_specs=[_full((1, n)), _full((1, n)), _full((1, 1)), _full((1, 1))],
        out_shape=[jax.ShapeDtypeStruct((1, n), f32),
                   jax.ShapeDtypeStruct((1, n), f32),
                   jax.ShapeDtypeStruct((1, 1), f32),
                   jax.ShapeDtypeStruct((1, 1), f32)],
    )(x2, g1_w2, g1_a, g1_a2, wc1_r)

    xvb1 = x2.astype(bf16)
    edge1, y1 = _edge_pass(Hb, s1, smax1, xvb1, g1_w3, g1_a2)
    aug1 = _node_pass(Hb, q1, qmax1, y1, edge1)

    xvb2, s2, q2, smax2, qmax2 = pl.pallas_call(
        _norm_prep_body,
        grid=(1,),
        in_specs=[_full((n, 2 * d)), _full((e, d)), _full((d, d)),
                  _full((d, d)), _full((2 * d, 1)), _full((2 * d, 1)),
                  _full((1, d))],
        out_specs=[_full((n, d)), _full((1, n)), _full((1, n)),
                   _full((1, 1)), _full((1, 1))],
        out_shape=[jax.ShapeDtypeStruct((n, d), bf16),
                   jax.ShapeDtypeStruct((1, n), f32),
                   jax.ShapeDtypeStruct((1, n), f32),
                   jax.ShapeDtypeStruct((1, 1), f32),
                   jax.ShapeDtypeStruct((1, 1), f32)],
    )(aug1, edge1, g2_w2, g2_w, g2_a, g2_a2, wc2_r)

    edge2, y2 = _edge_pass(Hb, s2, smax2, xvb2, g2_w3, g2_a2)
    aug2 = _node_pass(Hb, q2, qmax2, y2, edge2)

    out = pl.pallas_call(
        _norm_body,
        grid=(1,),
        in_specs=[_full((n, 2 * d)), _full((e, d))],
        out_specs=_full((n, d)),
        out_shape=jax.ShapeDtypeStruct((n, d), f32),
    )(aug2, edge2)
    return out.reshape(1, n, d)


# speedup vs baseline: 1.2889x; 1.0012x over previous
import jax
import jax.numpy as jnp
from jax import lax
from jax.experimental import pallas as pl

ALPHA = 0.2
N_NODE = 10000
N_EDGE = 1024
D = 128

TE2 = 256
TE3 = 128

f32 = jnp.float32
bf16 = jnp.bfloat16


def _lrelu(t):
    return jnp.where(t > 0, t, ALPHA * t)


def _sq(xa, a_ref, a2_ref, wc_ref):
    a_top = a_ref[0:D, :]
    a_bot = a_ref[D:2 * D, :]
    c = jnp.dot(wc_ref[...], a_top, preferred_element_type=f32)
    dot_rows = lambda v: lax.dot_general(v, xa, (((0,), (1,)), ((), ())),
                                         preferred_element_type=f32)
    s = _lrelu(c[0, 0] + dot_rows(a_bot))
    q = dot_rows(a2_ref[0:D, :])
    return s, q


def _cast_body(h_ref, hb_ref):
    hb_ref[...] = h_ref[...].astype(bf16)


def _prep_body(x_ref, w2_ref, a_ref, a2_ref, wc_ref,
               s_ref, q_ref, smax_ref, qmax_ref):
    xa = jnp.dot(x_ref[...], w2_ref[...], preferred_element_type=f32)
    s, q = _sq(xa, a_ref, a2_ref, wc_ref)
    s_ref[...] = s
    q_ref[...] = q
    smax_ref[...] = jnp.max(s).reshape(1, 1)
    qmax_ref[...] = jnp.max(q).reshape(1, 1)


def _edge_body(hb_ref, s_ref, smax_ref, xvb_ref, w3_ref, a2_ref,
               edge_ref, y_ref, ymax_ref):
    @pl.when(pl.program_id(0) == 0)
    def _():
        ymax_ref[...] = jnp.full((1, 1), -jnp.inf, f32)

    expw = jnp.exp(s_ref[...] - smax_ref[0, 0]).astype(bf16)
    w = hb_ref[...] * expw
    xvb = xvb_ref[...]
    num = jnp.dot(w, xvb, preferred_element_type=f32)
    den = jnp.sum(w.astype(f32), axis=1, keepdims=True)
    mx = jnp.sum(xvb.astype(f32), axis=0, keepdims=True) * (1.0 / N_NODE)
    edge = jnp.where(den > 0, num / den, mx)
    edge_ref[...] = edge
    w3a = jnp.dot(w3_ref[...], a2_ref[D:2 * D, :],
                  preferred_element_type=f32)
    y = jnp.dot(edge, w3a, preferred_element_type=f32)
    y_ref[...] = y
    ymax_ref[...] = jnp.maximum(ymax_ref[...], jnp.max(y).reshape(1, 1))


def _node_body(hb_ref, q_ref, qmax_ref, y_ref, ymax_ref, edge_ref, out_ref):
    @pl.when(pl.program_id(0) == 0)
    def _():
        out_ref[...] = jnp.zeros_like(out_ref)

    q = q_ref[...]
    qm = qmax_ref[0, 0]
    ym = ymax_ref[0, 0]
    a_row = jnp.exp(q - qm).astype(bf16)
    c_row = jnp.exp(ALPHA * q - qm).astype(bf16)
    y = y_ref[...]
    b_col = jnp.exp(y - ym).astype(bf16)
    d_col = jnp.exp(ALPHA * y - ym).astype(bf16)
    w = hb_ref[...] * jnp.maximum(b_col * a_row, d_col * c_row)
    aug = jnp.concatenate(
        [edge_ref[...].astype(bf16), jnp.ones((TE3, D), bf16)], axis=1)
    out_ref[...] += lax.dot_general(w, aug, (((0,), (0,)), ((), ())),
                                    preferred_element_type=f32)


def _norm_prep_body(aug_ref, edge_ref, w2_ref, w_ref, a_ref, a2_ref, wc_ref,
                    xvb_ref, s_ref, q_ref, smax_ref, qmax_ref):
    aug = aug_ref[...]
    num = aug[:, :D]
    den = aug[:, D:D + 1]
    emean = jnp.sum(edge_ref[...], axis=0, keepdims=True) * (1.0 / N_EDGE)
    node = jnp.where(den > 0, num / den, emean)
    h = jnp.where(node > 0, node, jnp.exp(node) - 1.0)
    xa = jnp.dot(h, w2_ref[...], preferred_element_type=f32)
    s, q = _sq(xa, a_ref, a2_ref, wc_ref)
    xvb_ref[...] = jnp.dot(h, w_ref[...], preferred_element_type=f32).astype(bf16)
    s_ref[...] = s
    q_ref[...] = q
    smax_ref[...] = jnp.max(s).reshape(1, 1)
    qmax_ref[...] = jnp.max(q).reshape(1, 1)


def _norm_body(aug_ref, edge_ref, out_ref):
    aug = aug_ref[...]
    num = aug[:, :D]
    den = aug[:, D:D + 1]
    emean = jnp.sum(edge_ref[...], axis=0, keepdims=True) * (1.0 / N_EDGE)
    out_ref[...] = jnp.where(den > 0, num / den, emean)


def _full(shape):
    nd = len(shape)
    return pl.BlockSpec(shape, lambda i: (0,) * nd)


def _rows(t):
    nd = len(t)
    return pl.BlockSpec(t, lambda i: (i,) + (0,) * (nd - 1))


def _edge_pass(Hb, s, smax, xvb, w3, a2_c):
    n, e, d = N_NODE, N_EDGE, D
    return pl.pallas_call(
        _edge_body,
        grid=(e // TE2,),
        in_specs=[_rows((TE2, n)), _full((1, n)), _full((1, 1)),
                  _full((n, d)), _full((d, d)), _full((2 * d, 1))],
        out_specs=[_rows((TE2, d)), _rows((TE2, 1)), _full((1, 1))],
        out_shape=[jax.ShapeDtypeStruct((e, d), f32),
                   jax.ShapeDtypeStruct((e, 1), f32),
                   jax.ShapeDtypeStruct((1, 1), f32)],
    )(Hb, s, smax, xvb, w3, a2_c)


def _node_pass(Hb, q, qmax, y, ymax, edge):
    n, e, d = N_NODE, N_EDGE, D
    return pl.pallas_call(
        _node_body,
        grid=(e // TE3,),
        in_specs=[_rows((TE3, n)), _full((1, n)), _full((1, 1)),
                  _rows((TE3, 1)), _full((1, 1)), _rows((TE3, d))],
        out_specs=_full((n, 2 * d)),
        out_shape=jax.ShapeDtypeStruct((n, 2 * d), f32),
    )(Hb, q, qmax, y, ymax, edge)


@jax.jit
def kernel(x, H, g1_w2, g1_w3, g1_wc, g1_a, g1_a2,
           g2_w, g2_w2, g2_w3, g2_wc, g2_a, g2_a2):
    n, e, d = N_NODE, N_EDGE, D
    x2 = x[0]
    H2 = H[0]
    wc1_r = g1_wc.reshape(1, d)
    wc2_r = g2_wc.reshape(1, d)

    Hb = pl.pallas_call(
        _cast_body,
        grid=(e // TE3,),
        in_specs=[_rows((TE3, n))],
        out_specs=_rows((TE3, n)),
        out_shape=jax.ShapeDtypeStruct((e, n), bf16),
    )(H2)

    s1, q1, smax1, qmax1 = pl.pallas_call(
        _prep_body,
        grid=(1,),
        in_specs=[_full((n, d)), _full((d, d)), _full((2 * d, 1)),
                  _full((2 * d, 1)), _full((1, d))],
        out_specs=[_full((1, n)), _full((1, n)), _full((1, 1)), _full((1, 1))],
        out_shape=[jax.ShapeDtypeStruct((1, n), f32),
                   jax.ShapeDtypeStruct((1, n), f32),
                   jax.ShapeDtypeStruct((1, 1), f32),
                   jax.ShapeDtypeStruct((1, 1), f32)],
    )(x2, g1_w2, g1_a, g1_a2, wc1_r)

    xvb1 = x2.astype(bf16)
    edge1, y1, ymax1 = _edge_pass(Hb, s1, smax1, xvb1, g1_w3, g1_a2)
    aug1 = _node_pass(Hb, q1, qmax1, y1, ymax1, edge1)

    xvb2, s2, q2, smax2, qmax2 = pl.pallas_call(
        _norm_prep_body,
        grid=(1,),
        in_specs=[_full((n, 2 * d)), _full((e, d)), _full((d, d)),
                  _full((d, d)), _full((2 * d, 1)), _full((2 * d, 1)),
                  _full((1, d))],
        out_specs=[_full((n, d)), _full((1, n)), _full((1, n)),
                   _full((1, 1)), _full((1, 1))],
        out_shape=[jax.ShapeDtypeStruct((n, d), bf16),
                   jax.ShapeDtypeStruct((1, n), f32),
                   jax.ShapeDtypeStruct((1, n), f32),
                   jax.ShapeDtypeStruct((1, 1), f32),
                   jax.ShapeDtypeStruct((1, 1), f32)],
    )(aug1, edge1, g2_w2, g2_w, g2_a, g2_a2, wc2_r)

    edge2, y2, ymax2 = _edge_pass(Hb, s2, smax2, xvb2, g2_w3, g2_a2)
    aug2 = _node_pass(Hb, q2, qmax2, y2, ymax2, edge2)

    out = pl.pallas_call(
        _norm_body,
        grid=(1,),
        in_specs=[_full((n, 2 * d)), _full((e, d))],
        out_specs=_full((n, d)),
        out_shape=jax.ShapeDtypeStruct((n, d), f32),
    )(aug2, edge2)
    return out.reshape(1, n, d)


# speedup vs baseline: 1.3309x; 1.0326x over previous
import jax
import jax.numpy as jnp
from jax import lax
from jax.experimental import pallas as pl

ALPHA = 0.2
N_NODE = 10000
N_EDGE = 1024
D = 128

TE2 = 256
TE3 = 128

f32 = jnp.float32
bf16 = jnp.bfloat16


def _lrelu(t):
    return jnp.where(t > 0, t, ALPHA * t)


def _rows_from_xa(xa, a_ref, a2_ref, wc_ref):
    a_top = a_ref[0:D, :]
    a_bot = a_ref[D:2 * D, :]
    c = jnp.dot(wc_ref[...], a_top, preferred_element_type=f32)
    dot_rows = lambda v: lax.dot_general(v, xa, (((0,), (1,)), ((), ())),
                                         preferred_element_type=f32)
    s = _lrelu(c[0, 0] + dot_rows(a_bot))
    q = dot_rows(a2_ref[0:D, :])
    expw = jnp.exp(s - jnp.max(s)).astype(bf16)
    qm = jnp.max(q)
    arow = jnp.exp(q - qm).astype(bf16)
    crow = jnp.exp(ALPHA * q - qm).astype(bf16)
    return expw, arow, crow


def _aug_ones(m, t):
    return jnp.concatenate([m.astype(bf16), jnp.ones((t, D), bf16)], axis=1)


def _cast_body(h_ref, hb_ref):
    hb_ref[...] = h_ref[...].astype(bf16)


def _prep_body(x_ref, w2_ref, a_ref, a2_ref, wc_ref,
               xvb_ref, expw_ref, arow_ref, crow_ref):
    x = x_ref[...]
    xa = jnp.dot(x, w2_ref[...], preferred_element_type=f32)
    expw, arow, crow = _rows_from_xa(xa, a_ref, a2_ref, wc_ref)
    expw_ref[...] = expw
    arow_ref[...] = arow
    crow_ref[...] = crow
    xvb_ref[...] = _aug_ones(x, N_NODE)


def _edge_body(hb_ref, expw_ref, xvb_ref, w3_ref, a2_ref,
               edge_ref, eaug_ref, y_ref, ymax_ref):
    @pl.when(pl.program_id(0) == 0)
    def _():
        ymax_ref[...] = jnp.full((1, 1), -jnp.inf, f32)

    w = hb_ref[...] * expw_ref[...]
    numaug = jnp.dot(w, xvb_ref[...], preferred_element_type=f32)
    num = numaug[:, :D]
    den = numaug[:, D:D + 1]
    mx = jnp.sum(xvb_ref[...][:, :D].astype(f32), axis=0, keepdims=True) \
        * (1.0 / N_NODE)
    edge = jnp.where(den > 0, num / den, mx)
    edge_ref[...] = edge
    eaug_ref[...] = _aug_ones(edge, TE2)
    w3a = jnp.dot(w3_ref[...], a2_ref[D:2 * D, :],
                  preferred_element_type=f32)
    y = jnp.dot(edge, w3a, preferred_element_type=f32)
    y_ref[...] = y
    ymax_ref[...] = jnp.maximum(ymax_ref[...], jnp.max(y).reshape(1, 1))


def _node_body(hb_ref, arow_ref, crow_ref, y_ref, ymax_ref, eaug_ref, out_ref):
    @pl.when(pl.program_id(0) == 0)
    def _():
        out_ref[...] = jnp.zeros_like(out_ref)

    ym = ymax_ref[0, 0]
    y = y_ref[...]
    b_col = jnp.exp(y - ym).astype(bf16)
    d_col = jnp.exp(ALPHA * y - ym).astype(bf16)
    w = hb_ref[...] * jnp.maximum(b_col * arow_ref[...], d_col * crow_ref[...])
    out_ref[...] += lax.dot_general(w, eaug_ref[...], (((0,), (0,)), ((), ())),
                                    preferred_element_type=f32)


def _norm_prep_body(aug_ref, edge_ref, w2_ref, w_ref, a_ref, a2_ref, wc_ref,
                    xvb_ref, expw_ref, arow_ref, crow_ref):
    aug = aug_ref[...]
    num = aug[:, :D]
    den = aug[:, D:D + 1]
    emean = jnp.sum(edge_ref[...], axis=0, keepdims=True) * (1.0 / N_EDGE)
    node = jnp.where(den > 0, num / den, emean)
    h = jnp.where(node > 0, node, jnp.exp(node) - 1.0)
    xa = jnp.dot(h, w2_ref[...], preferred_element_type=f32)
    expw, arow, crow = _rows_from_xa(xa, a_ref, a2_ref, wc_ref)
    expw_ref[...] = expw
    arow_ref[...] = arow
    crow_ref[...] = crow
    xv = jnp.dot(h, w_ref[...], preferred_element_type=f32)
    xvb_ref[...] = _aug_ones(xv, N_NODE)


def _norm_body(aug_ref, edge_ref, out_ref):
    aug = aug_ref[...]
    num = aug[:, :D]
    den = aug[:, D:D + 1]
    emean = jnp.sum(edge_ref[...], axis=0, keepdims=True) * (1.0 / N_EDGE)
    out_ref[...] = jnp.where(den > 0, num / den, emean)


def _full(shape):
    nd = len(shape)
    return pl.BlockSpec(shape, lambda i: (0,) * nd)


def _rows(t):
    nd = len(t)
    return pl.BlockSpec(t, lambda i: (i,) + (0,) * (nd - 1))


def _edge_pass(Hb, expw, xvb, w3, a2_c):
    n, e, d = N_NODE, N_EDGE, D
    return pl.pallas_call(
        _edge_body,
        grid=(e // TE2,),
        in_specs=[_rows((TE2, n)), _full((1, n)), _full((n, 2 * d)),
                  _full((d, d)), _full((2 * d, 1))],
        out_specs=[_rows((TE2, d)), _rows((TE2, 2 * d)), _rows((TE2, 1)),
                   _full((1, 1))],
        out_shape=[jax.ShapeDtypeStruct((e, d), f32),
                   jax.ShapeDtypeStruct((e, 2 * d), bf16),
                   jax.ShapeDtypeStruct((e, 1), f32),
                   jax.ShapeDtypeStruct((1, 1), f32)],
    )(Hb, expw, xvb, w3, a2_c)


def _node_pass(Hb, arow, crow, y, ymax, eaug):
    n, e, d = N_NODE, N_EDGE, D
    return pl.pallas_call(
        _node_body,
        grid=(e // TE3,),
        in_specs=[_rows((TE3, n)), _full((1, n)), _full((1, n)),
                  _rows((TE3, 1)), _full((1, 1)), _rows((TE3, 2 * d))],
        out_specs=_full((n, 2 * d)),
        out_shape=jax.ShapeDtypeStruct((n, 2 * d), f32),
    )(Hb, arow, crow, y, ymax, eaug)


@jax.jit
def kernel(x, H, g1_w2, g1_w3, g1_wc, g1_a, g1_a2,
           g2_w, g2_w2, g2_w3, g2_wc, g2_a, g2_a2):
    n, e, d = N_NODE, N_EDGE, D
    x2 = x[0]
    H2 = H[0]
    wc1_r = g1_wc.reshape(1, d)
    wc2_r = g2_wc.reshape(1, d)

    Hb = pl.pallas_call(
        _cast_body,
        grid=(e // TE3,),
        in_specs=[_rows((TE3, n))],
        out_specs=_rows((TE3, n)),
        out_shape=jax.ShapeDtypeStruct((e, n), bf16),
    )(H2)

    row = jax.ShapeDtypeStruct((1, n), bf16)
    xvb1, expw1, arow1, crow1 = pl.pallas_call(
        _prep_body,
        grid=(1,),
        in_specs=[_full((n, d)), _full((d, d)), _full((2 * d, 1)),
                  _full((2 * d, 1)), _full((1, d))],
        out_specs=[_full((n, 2 * d)), _full((1, n)), _full((1, n)),
                   _full((1, n))],
        out_shape=[jax.ShapeDtypeStruct((n, 2 * d), bf16), row, row, row],
    )(x2, g1_w2, g1_a, g1_a2, wc1_r)

    edge1, eaug1, y1, ymax1 = _edge_pass(Hb, expw1, xvb1, g1_w3, g1_a2)
    aug1 = _node_pass(Hb, arow1, crow1, y1, ymax1, eaug1)

    xvb2, expw2, arow2, crow2 = pl.pallas_call(
        _norm_prep_body,
        grid=(1,),
        in_specs=[_full((n, 2 * d)), _full((e, d)), _full((d, d)),
                  _full((d, d)), _full((2 * d, 1)), _full((2 * d, 1)),
                  _full((1, d))],
        out_specs=[_full((n, 2 * d)), _full((1, n)), _full((1, n)),
                   _full((1, n))],
        out_shape=[jax.ShapeDtypeStruct((n, 2 * d), bf16), row, row, row],
    )(aug1, edge1, g2_w2, g2_w, g2_a, g2_a2, wc2_r)

    edge2, eaug2, y2, ymax2 = _edge_pass(Hb, expw2, xvb2, g2_w3, g2_a2)
    aug2 = _node_pass(Hb, arow2, crow2, y2, ymax2, eaug2)

    out = pl.pallas_call(
        _norm_body,
        grid=(1,),
        in_specs=[_full((n, 2 * d)), _full((e, d))],
        out_specs=_full((n, d)),
        out_shape=jax.ShapeDtypeStruct((n, d), f32),
    )(aug2, edge2)
    return out.reshape(1, n, d)
